# baseline (speedup 1.0000x reference)
import functools

import jax
import jax.numpy as jnp
from jax import lax
from jax.experimental import pallas as pl
from jax.experimental.pallas import tpu as pltpu
from jax.experimental.pallas import tpu_sc as plsc

ROWS = 128
COLS = 32768
LANES = 16
CHUNKS = COLS // LANES
GSIZE = 8
NGROUPS = CHUNKS // GSIZE
NC, NS = 2, 16
NW = NC * NS
ROWS_PER_W = ROWS // NW
NEG = -1e30


def _iota16():
    return lax.broadcasted_iota(jnp.int32, (LANES,), 0)


def _insert(carry, x, tag, exact=False):
    m1, m2, m3, i1, i2, i3 = carry
    if exact:
        dup = ((x == m1) & (tag == i1)) | ((x == m2) & (tag == i2)) | (
            (x == m3) & (tag == i3))
        keep = ~dup
        t1 = ((x > m1) | ((x == m1) & (tag < i1))) & keep
        t2 = ((x > m2) | ((x == m2) & (tag < i2))) & keep
        t3 = ((x > m3) | ((x == m3) & (tag < i3))) & keep
    else:
        t1 = x > m1
        t2 = x > m2
        t3 = x > m3
    n_m3 = jnp.where(t2, m2, jnp.where(t3, x, m3))
    n_i3 = jnp.where(t2, i2, jnp.where(t3, tag, i3))
    n_m2 = jnp.where(t1, m1, jnp.where(t2, x, m2))
    n_i2 = jnp.where(t1, i1, jnp.where(t2, tag, i2))
    n_m1 = jnp.where(t1, x, m1)
    n_i1 = jnp.where(t1, tag, i1)
    return n_m1, n_m2, n_m3, n_i1, n_i2, n_i3


def _fresh_carry():
    negv = jnp.full((LANES,), NEG, jnp.float32)
    zv = jnp.zeros((LANES,), jnp.int32)
    return (negv, negv, negv, zv, zv, zv)


def _merge_pop(carry):
    m1, m2, m3, i1, i2, i3 = carry
    r = jnp.max(m1)
    sel = m1 == r
    colv = i1 * LANES + _iota16()
    col = jnp.min(jnp.where(sel, colv, jnp.int32(2147483647)))
    lm = sel & (colv == col)
    n_m1 = jnp.where(lm, m2, m1)
    n_i1 = jnp.where(lm, i2, i1)
    n_m2 = jnp.where(lm, m3, m2)
    n_i2 = jnp.where(lm, i3, i2)
    n_m3 = jnp.where(lm, jnp.float32(NEG), m3)
    return (r, col), (n_m1, n_m2, n_m3, n_i1, n_i2, i3)


def _sc_body(scores_hbm, out_hbm, in_v, out_v, in_sems, out_sem):
    wid = lax.axis_index("s") * NC + lax.axis_index("c")
    iota = _iota16()
    zeros16 = jnp.zeros((LANES,), jnp.float32)

    row0 = wid * ROWS_PER_W
    pltpu.async_copy(scores_hbm.at[row0], in_v.at[0], in_sems.at[0])
    pltpu.async_copy(scores_hbm.at[row0 + 1], in_v.at[1], in_sems.at[1])

    def zbody(i, _):
        out_v[pl.ds(i * LANES, LANES)] = zeros16
        return 0

    lax.fori_loop(0, CHUNKS, zbody, 0, unroll=8)

    prev_idx0 = iota
    mask3 = iota < 3

    def row_body(r, prev_idx):
        row = row0 + r
        buf = r % 2
        pltpu.make_async_copy(
            scores_hbm.at[row], in_v.at[buf], in_sems.at[buf]
        ).wait()

        @pl.when((r > 0) & (r < ROWS_PER_W - 1))
        def _prefetch():
            pltpu.async_copy(
                scores_hbm.at[row + 1], in_v.at[1 - buf], in_sems.at[1 - buf]
            )

        def scan_group(g, carry):
            base = g * (GSIZE * LANES)
            gm = in_v[buf, pl.ds(base, LANES)]
            for u in range(1, GSIZE):
                gm = jnp.maximum(gm, in_v[buf, pl.ds(base + u * LANES, LANES)])
            return _insert(carry, gm, g)

        sc = lax.fori_loop(0, NGROUPS, scan_group, _fresh_carry(), unroll=2)

        gids = []
        for _ in range(5):
            (_, gcol), sc = _merge_pop(sc)
            gids.append(lax.shift_right_logical(gcol, 4))

        fc = _fresh_carry()
        for gid in gids:
            def rescan_chunk(u, carry, gid=gid):
                x = in_v[buf, pl.ds(gid * (GSIZE * LANES) + u * LANES, LANES)]
                return _insert(carry, x, gid * GSIZE + u, exact=True)

            fc = lax.fori_loop(0, GSIZE, rescan_chunk, fc)

        (v1, col1), fc = _merge_pop(fc)
        (v2, col2), fc = _merge_pop(fc)
        (v3, col3), fc = _merge_pop(fc)

        vals = jnp.where(
            iota == 0, v1, jnp.where(iota == 1, v2, jnp.where(iota == 2, v3, jnp.float32(NEG)))
        )
        e = jnp.exp(vals - v1)
        p = e / jnp.sum(e)

        idx = jnp.where(iota == 0, col1, jnp.where(iota == 1, col2, jnp.where(iota == 2, col3, jnp.int32(0))))

        @pl.when(r > 0)
        def _clear_prev():
            pltpu.make_async_copy(out_v, out_hbm.at[row - 1], out_sem).wait()
            plsc.store_scatter(out_v, [prev_idx], zeros16, mask=mask3)

        plsc.store_scatter(out_v, [idx], p, mask=mask3)
        pltpu.async_copy(out_v, out_hbm.at[row], out_sem)
        return idx

    lax.fori_loop(0, ROWS_PER_W, row_body, prev_idx0)

    pltpu.make_async_copy(
        out_v, out_hbm.at[row0 + ROWS_PER_W - 1], out_sem
    ).wait()


def kernel(scores):
    mesh = plsc.VectorSubcoreMesh(
        core_axis_name="c", subcore_axis_name="s", num_cores=NC, num_subcores=NS
    )
    fn = pl.kernel(
        _sc_body,
        out_type=jax.ShapeDtypeStruct((ROWS, COLS), jnp.float32),
        mesh=mesh,
        compiler_params=pltpu.CompilerParams(needs_layout_passes=False),
        scratch_types=[
            pltpu.VMEM((2, COLS), jnp.float32),
            pltpu.VMEM((COLS,), jnp.float32),
            pltpu.SemaphoreType.DMA((2,)),
            pltpu.SemaphoreType.DMA,
        ],
    )
    return fn(scores)


# speedup vs baseline: 1.0161x; 1.0161x over previous
import functools

import jax
import jax.numpy as jnp
from jax import lax
from jax.experimental import pallas as pl
from jax.experimental.pallas import tpu as pltpu
from jax.experimental.pallas import tpu_sc as plsc

ROWS = 128
COLS = 32768
LANES = 16
CHUNKS = COLS // LANES
GSIZE = 8
NGROUPS = CHUNKS // GSIZE
NC, NS = 2, 16
NW = NC * NS
ROWS_PER_W = ROWS // NW
NEG = -1e30


def _iota16():
    return lax.broadcasted_iota(jnp.int32, (LANES,), 0)


def _insert(carry, x, tag, exact=False):
    m1, m2, m3, i1, i2, i3 = carry
    if exact:
        dup = ((x == m1) & (tag == i1)) | ((x == m2) & (tag == i2)) | (
            (x == m3) & (tag == i3))
        keep = ~dup
        t1 = ((x > m1) | ((x == m1) & (tag < i1))) & keep
        t2 = ((x > m2) | ((x == m2) & (tag < i2))) & keep
        t3 = ((x > m3) | ((x == m3) & (tag < i3))) & keep
    else:
        t1 = x > m1
        t2 = x > m2
        t3 = x > m3
    n_m3 = jnp.where(t2, m2, jnp.where(t3, x, m3))
    n_i3 = jnp.where(t2, i2, jnp.where(t3, tag, i3))
    n_m2 = jnp.where(t1, m1, jnp.where(t2, x, m2))
    n_i2 = jnp.where(t1, i1, jnp.where(t2, tag, i2))
    n_m1 = jnp.where(t1, x, m1)
    n_i1 = jnp.where(t1, tag, i1)
    return n_m1, n_m2, n_m3, n_i1, n_i2, n_i3


def _fresh_carry():
    negv = jnp.full((LANES,), NEG, jnp.float32)
    zv = jnp.zeros((LANES,), jnp.int32)
    return (negv, negv, negv, zv, zv, zv)


def _merge_pop(carry):
    m1, m2, m3, i1, i2, i3 = carry
    r = jnp.max(m1)
    sel = m1 == r
    colv = i1 * LANES + _iota16()
    col = jnp.min(jnp.where(sel, colv, jnp.int32(2147483647)))
    lm = sel & (colv == col)
    n_m1 = jnp.where(lm, m2, m1)
    n_i1 = jnp.where(lm, i2, i1)
    n_m2 = jnp.where(lm, m3, m2)
    n_i2 = jnp.where(lm, i3, i2)
    n_m3 = jnp.where(lm, jnp.float32(NEG), m3)
    return (r, col), (n_m1, n_m2, n_m3, n_i1, n_i2, i3)


def _sc_body(scores_hbm, out_hbm, in_v, out_v, in_sems, out_sem):
    wid = lax.axis_index("s") * NC + lax.axis_index("c")
    iota = _iota16()
    zeros16 = jnp.zeros((LANES,), jnp.float32)

    row0 = wid * ROWS_PER_W
    HALF = COLS // 2
    for b in (0, 1):
        pltpu.async_copy(
            scores_hbm.at[row0 + b, pl.ds(0, HALF)],
            in_v.at[b, pl.ds(0, HALF)], in_sems.at[b, 0])
        pltpu.async_copy(
            scores_hbm.at[row0 + b, pl.ds(HALF, HALF)],
            in_v.at[b, pl.ds(HALF, HALF)], in_sems.at[b, 1])

    def zbody(i, _):
        out_v[pl.ds(i * LANES, LANES)] = zeros16
        return 0

    lax.fori_loop(0, CHUNKS, zbody, 0, unroll=8)

    prev_idx0 = iota
    mask3 = iota < 3

    def row_body(r, prev_idx):
        row = row0 + r
        buf = r % 2
        HALF = COLS // 2
        pltpu.make_async_copy(
            scores_hbm.at[row, pl.ds(0, HALF)],
            in_v.at[buf, pl.ds(0, HALF)], in_sems.at[buf, 0]
        ).wait()

        @pl.when((r > 0) & (r < ROWS_PER_W - 1))
        def _prefetch():
            pltpu.async_copy(
                scores_hbm.at[row + 1, pl.ds(0, HALF)],
                in_v.at[1 - buf, pl.ds(0, HALF)], in_sems.at[1 - buf, 0])
            pltpu.async_copy(
                scores_hbm.at[row + 1, pl.ds(HALF, HALF)],
                in_v.at[1 - buf, pl.ds(HALF, HALF)], in_sems.at[1 - buf, 1])

        def scan_group(g, carry):
            base = g * (GSIZE * LANES)
            gm = in_v[buf, pl.ds(base, LANES)]
            for u in range(1, GSIZE):
                gm = jnp.maximum(gm, in_v[buf, pl.ds(base + u * LANES, LANES)])
            return _insert(carry, gm, g)

        sc = lax.fori_loop(0, NGROUPS // 2, scan_group, _fresh_carry(), unroll=2)
        pltpu.make_async_copy(
            scores_hbm.at[row, pl.ds(HALF, HALF)],
            in_v.at[buf, pl.ds(HALF, HALF)], in_sems.at[buf, 1]
        ).wait()
        sc = lax.fori_loop(NGROUPS // 2, NGROUPS, scan_group, sc, unroll=2)

        gids = []
        for _ in range(5):
            (_, gcol), sc = _merge_pop(sc)
            gids.append(lax.shift_right_logical(gcol, 4))

        fc = _fresh_carry()
        for gid in gids:
            def rescan_chunk(u, carry, gid=gid):
                x = in_v[buf, pl.ds(gid * (GSIZE * LANES) + u * LANES, LANES)]
                return _insert(carry, x, gid * GSIZE + u, exact=True)

            fc = lax.fori_loop(0, GSIZE, rescan_chunk, fc)

        (v1, col1), fc = _merge_pop(fc)
        (v2, col2), fc = _merge_pop(fc)
        (v3, col3), fc = _merge_pop(fc)

        vals = jnp.where(
            iota == 0, v1, jnp.where(iota == 1, v2, jnp.where(iota == 2, v3, jnp.float32(NEG)))
        )
        e = jnp.exp(vals - v1)
        p = e / jnp.sum(e)

        idx = jnp.where(iota == 0, col1, jnp.where(iota == 1, col2, jnp.where(iota == 2, col3, jnp.int32(0))))

        @pl.when(r > 0)
        def _clear_prev():
            pltpu.make_async_copy(out_v, out_hbm.at[row - 1], out_sem).wait()
            plsc.store_scatter(out_v, [prev_idx], zeros16, mask=mask3)

        plsc.store_scatter(out_v, [idx], p, mask=mask3)
        pltpu.async_copy(out_v, out_hbm.at[row], out_sem)
        return idx

    lax.fori_loop(0, ROWS_PER_W, row_body, prev_idx0)

    pltpu.make_async_copy(
        out_v, out_hbm.at[row0 + ROWS_PER_W - 1], out_sem
    ).wait()


def kernel(scores):
    mesh = plsc.VectorSubcoreMesh(
        core_axis_name="c", subcore_axis_name="s", num_cores=NC, num_subcores=NS
    )
    fn = pl.kernel(
        _sc_body,
        out_type=jax.ShapeDtypeStruct((ROWS, COLS), jnp.float32),
        mesh=mesh,
        compiler_params=pltpu.CompilerParams(needs_layout_passes=False),
        scratch_types=[
            pltpu.VMEM((2, COLS), jnp.float32),
            pltpu.VMEM((COLS,), jnp.float32),
            pltpu.SemaphoreType.DMA((2, 2)),
            pltpu.SemaphoreType.DMA,
        ],
    )
    return fn(scores)
